# baseline (speedup 1.0000x reference)
import functools

import jax
import jax.numpy as jnp
from jax import lax
from jax.experimental import pallas as pl
from jax.experimental.pallas import tpu as pltpu
from jax.experimental.pallas import tpu_sc as plsc

_NUM_EMB = 1000000
_DIM = 64
_SCALE = float(_DIM) ** 0.5

_NC, _NS = 2, 16
_NW = _NC * _NS
_B = 16384 * 50
_G = 128
_BPW = _B // _NW
_NG = _BPW // _G


def _body(idx_hbm, w_hbm, out_hbm, idx_v, gbuf, sem_i, sem_g):
    wid = lax.axis_index("s") * _NC + lax.axis_index("c")
    pltpu.async_copy(idx_hbm.at[pl.ds(wid * _NG, _NG)], idx_v, sem_i).wait()
    row0 = wid * _BPW

    @pl.loop(0, _NG)
    def _group(g):
        pltpu.async_copy(w_hbm.at[idx_v.at[g]], gbuf, sem_g).wait()

        @pl.loop(0, _G)
        def _row(r):
            for c in range(_DIM // 16):
                sl = pl.ds(c * 16, 16)
                gbuf[r, sl] = gbuf[r, sl] * _SCALE

        pltpu.sync_copy(gbuf, out_hbm.at[pl.ds(row0 + g * _G, _G)])


@jax.jit
def _emb(idx2d, weight):
    mesh = plsc.VectorSubcoreMesh(core_axis_name="c", subcore_axis_name="s")
    return pl.kernel(
        _body,
        out_type=jax.ShapeDtypeStruct((_B, _DIM), jnp.float32),
        mesh=mesh,
        compiler_params=pltpu.CompilerParams(use_tc_tiling_on_sc=False),
        scratch_types=[
            pltpu.VMEM((_NG, _G), jnp.int32),
            pltpu.VMEM((_G, _DIM), jnp.float32),
            pltpu.SemaphoreType.DMA,
            pltpu.SemaphoreType.DMA,
        ],
    )(idx2d, weight)


def kernel(input, weight):
    idx2d = input.reshape(_B // _G, _G).astype(jnp.int32)
    out = _emb(idx2d, weight)
    return out.reshape(input.shape[0], input.shape[1], _DIM)


# speedup vs baseline: 1.0976x; 1.0976x over previous
import jax
import jax.numpy as jnp
from jax import lax
from jax.experimental import pallas as pl
from jax.experimental.pallas import tpu as pltpu
from jax.experimental.pallas import tpu_sc as plsc

_NUM_EMB = 1000000
_DIM = 64
_SCALE = float(_DIM) ** 0.5

_NC, _NS = 2, 16
_NW = _NC * _NS
_B = 16384 * 50
_G = 128
_BPW = _B // _NW
_NG = _BPW // _G
_NBUF = 4


def _scale_group(src, dst):
    @pl.loop(0, _G, unroll=8)
    def _row(r):
        for c in range(_DIM // 16):
            sl = pl.ds(c * 16, 16)
            dst[r, sl] = src[r, sl] * _SCALE


def _body(idx_hbm, w_hbm, out_hbm, idx_v, gb, sb, *sems):
    sg, ss = sems[:_NBUF], sems[_NBUF:]
    wid = lax.axis_index("s") * _NC + lax.axis_index("c")
    pltpu.async_copy(idx_hbm.at[pl.ds(wid * _NG, _NG)], idx_v, ss[0]).wait()
    row0 = wid * _BPW

    for b in range(_NBUF):
        pltpu.async_copy(w_hbm.at[idx_v.at[b]], gb.at[b], sg[b])

    @pl.loop(0, _NG, step=_NBUF)
    def _grp(g0):
        for b in range(_NBUF):
            g = g0 + b
            pltpu.make_async_copy(w_hbm.at[idx_v.at[b]], gb.at[b], sg[b]).wait()
            @pl.when(g0 > 0)
            def _():
                pltpu.make_async_copy(
                    sb.at[b], out_hbm.at[pl.ds(row0, _G)], ss[b]).wait()
            _scale_group(gb.at[b], sb.at[b])
            @pl.when(g0 < _NG - _NBUF)
            def _():
                pltpu.async_copy(w_hbm.at[idx_v.at[g + _NBUF]], gb.at[b], sg[b])
            pltpu.async_copy(
                sb.at[b], out_hbm.at[pl.ds(row0 + g * _G, _G)], ss[b])

    for b in range(_NBUF):
        pltpu.make_async_copy(sb.at[b], out_hbm.at[pl.ds(row0, _G)], ss[b]).wait()


@jax.jit
def _emb(idx2d, weight):
    mesh = plsc.VectorSubcoreMesh(core_axis_name="c", subcore_axis_name="s")
    return pl.kernel(
        _body,
        out_type=jax.ShapeDtypeStruct((_B, _DIM), jnp.float32),
        mesh=mesh,
        compiler_params=pltpu.CompilerParams(use_tc_tiling_on_sc=False),
        scratch_types=(
            [pltpu.VMEM((_NG, _G), jnp.int32),
             pltpu.VMEM((_NBUF, _G, _DIM), jnp.float32),
             pltpu.VMEM((_NBUF, _G, _DIM), jnp.float32)]
            + [pltpu.SemaphoreType.DMA] * (2 * _NBUF)
        ),
    )(idx2d, weight)


def kernel(input, weight):
    idx2d = input.reshape(_B // _G, _G).astype(jnp.int32)
    out = _emb(idx2d, weight)
    return out.reshape(input.shape[0], input.shape[1], _DIM)
